# Initial kernel scaffold; baseline (speedup 1.0000x reference)
import jax
import jax.numpy as jnp
from jax.experimental import pallas as pl


def kernel(x, edge_index, W1_l, b1, W1_r, W2_l, b2, W2_r):
    raise NotImplementedError("write your pallas kernel here")


# speedup vs baseline: 3.2924x; 3.2924x over previous
import functools

import jax
import jax.numpy as jnp
from jax import lax
from jax.experimental import pallas as pl
from jax.experimental.pallas import tpu as pltpu
from jax.experimental.pallas import tpu_sc as plsc

N = 10000
E = 160000
D = 256
H = 128
NC, NS = 2, 16
EPT = E // NS
K = 80
NCHUNK = EPT // K
RPT = 640
LAST = N - (NS - 1) * RPT
ZR = 80
EPT2 = E // (NC * NS)
K2 = 40
NCHUNK2 = EPT2 // K2

_f32 = jnp.float32
_mesh = plsc.VectorSubcoreMesh(core_axis_name="c", subcore_axis_name="s")


def _row_ranges(sid):
  return sid * RPT


@functools.partial(
    pl.kernel, mesh=_mesh,
    out_type=jax.ShapeDtypeStruct((NC, N, H), _f32),
    scratch_types=[
        pltpu.VMEM((K,), jnp.int32),
        pltpu.VMEM((K,), jnp.int32),
        pltpu.VMEM((K, H), _f32),
        pltpu.VMEM((ZR, H), _f32),
        pltpu.VMEM_SHARED((N, H), _f32),
    ])
def _agg(table_hbm, src_hbm, dst_hbm, out_hbm, srcb, dstb, rows, zrows, acc):
  cid = lax.axis_index("c")
  sid = lax.axis_index("s")
  z16 = jnp.zeros((1, 16), _f32)

  @pl.loop(0, ZR)
  def _(r):
    @pl.loop(0, H // 16)
    def _(c):
      zrows.at[pl.ds(r, 1), pl.ds(c * 16, 16)][...] = z16

  row0 = sid * RPT

  @pl.when(sid < NS - 1)
  def _():
    for j in range(RPT // ZR):
      pltpu.sync_copy(zrows, acc.at[pl.ds(row0 + j * ZR, ZR)])

  @pl.when(sid == NS - 1)
  def _():
    for j in range(LAST // ZR):
      pltpu.sync_copy(zrows, acc.at[pl.ds(row0 + j * ZR, ZR)])

  plsc.subcore_barrier()

  e0 = sid * EPT

  @pl.loop(0, NCHUNK)
  def _(i):
    base = e0 + i * K
    pltpu.sync_copy(src_hbm.at[pl.ds(base, K)], srcb)
    pltpu.sync_copy(dst_hbm.at[pl.ds(base, K)], dstb)
    pltpu.sync_copy(table_hbm.at[cid].at[srcb], rows)
    pltpu.sync_copy(rows, acc.at[dstb], add=True)

  plsc.subcore_barrier()

  @pl.when(sid < NS - 1)
  def _():
    pltpu.sync_copy(acc.at[pl.ds(row0, RPT)],
                    out_hbm.at[cid].at[pl.ds(row0, RPT)])

  @pl.when(sid == NS - 1)
  def _():
    pltpu.sync_copy(acc.at[pl.ds(row0, LAST)],
                    out_hbm.at[cid].at[pl.ds(row0, LAST)])


@functools.partial(
    pl.kernel, mesh=_mesh,
    out_type=jax.ShapeDtypeStruct((NC, N, H), _f32),
    scratch_types=[
        pltpu.VMEM((K2,), jnp.int32),
        pltpu.VMEM((K2, H), _f32),
        pltpu.VMEM((ZR, H), _f32),
        pltpu.VMEM_SHARED((N, H), _f32),
    ])
def _count(dst_hbm, out_hbm, dstb, oneb, zrows, cntacc):
  cid = lax.axis_index("c")
  sid = lax.axis_index("s")
  z16 = jnp.zeros((1, 16), _f32)
  one16 = jnp.ones((1, 16), _f32)

  @pl.loop(0, ZR)
  def _(r):
    @pl.loop(0, H // 16)
    def _(c):
      zrows.at[pl.ds(r, 1), pl.ds(c * 16, 16)][...] = z16

  @pl.loop(0, K2)
  def _(r):
    @pl.loop(0, H // 16)
    def _(c):
      oneb.at[pl.ds(r, 1), pl.ds(c * 16, 16)][...] = one16

  row0 = sid * RPT

  @pl.when(sid < NS - 1)
  def _():
    for j in range(RPT // ZR):
      pltpu.sync_copy(zrows, cntacc.at[pl.ds(row0 + j * ZR, ZR)])

  @pl.when(sid == NS - 1)
  def _():
    for j in range(LAST // ZR):
      pltpu.sync_copy(zrows, cntacc.at[pl.ds(row0 + j * ZR, ZR)])

  plsc.subcore_barrier()

  e0 = (cid * NS + sid) * EPT2

  @pl.loop(0, NCHUNK2)
  def _(i):
    pltpu.sync_copy(dst_hbm.at[pl.ds(e0 + i * K2, K2)], dstb)
    pltpu.sync_copy(oneb, cntacc.at[dstb], add=True)

  plsc.subcore_barrier()

  @pl.when(sid < NS - 1)
  def _():
    pltpu.sync_copy(cntacc.at[pl.ds(row0, RPT)],
                    out_hbm.at[cid].at[pl.ds(row0, RPT)])

  @pl.when(sid == NS - 1)
  def _():
    pltpu.sync_copy(cntacc.at[pl.ds(row0, LAST)],
                    out_hbm.at[cid].at[pl.ds(row0, LAST)])


R = 400
_CT = (((1,), (1,)), ((), ()))


def _dense_body(agg_ref, cnt_ref, x_ref, wl_ref, wr_ref, b_ref):
  cnt = cnt_ref[0][:, 0:1] + cnt_ref[1][:, 0:1]
  inv = 1.0 / jnp.maximum(cnt, 1.0)
  wl = wl_ref[...]
  wr = wr_ref[...]
  z = lax.dot_general(agg_ref[0] * inv, wl[:, :H], _CT,
                      preferred_element_type=_f32)
  z += lax.dot_general(agg_ref[1] * inv, wl[:, H:], _CT,
                       preferred_element_type=_f32)
  z += lax.dot_general(x_ref[0], wr[:, :H], _CT, preferred_element_type=_f32)
  z += lax.dot_general(x_ref[1], wr[:, H:], _CT, preferred_element_type=_f32)
  return z + b_ref[...]


def _dense1_kernel(agg_ref, cnt_ref, x_ref, wl_ref, wr_ref, b_ref, out_ref):
  h = jnp.maximum(_dense_body(agg_ref, cnt_ref, x_ref, wl_ref, wr_ref, b_ref),
                  0.0)
  out_ref[0] = h[:, :H]
  out_ref[1] = h[:, H:]


def _dense2_kernel(agg_ref, cnt_ref, x_ref, wl_ref, wr_ref, b_ref, out_ref):
  z = _dense_body(agg_ref, cnt_ref, x_ref, wl_ref, wr_ref, b_ref)
  nrm = jnp.sqrt(jnp.sum(z * z, axis=1, keepdims=True))
  out_ref[...] = z / jnp.maximum(nrm, 1e-12)


_split_spec = pl.BlockSpec((2, R, H), lambda i: (0, i, 0))
_in_specs = [
    _split_spec,
    _split_spec,
    _split_spec,
    pl.BlockSpec((D, D), lambda i: (0, 0)),
    pl.BlockSpec((D, D), lambda i: (0, 0)),
    pl.BlockSpec((1, D), lambda i: (0, 0)),
]

_dense1 = pl.pallas_call(
    _dense1_kernel,
    grid=(N // R,),
    in_specs=_in_specs,
    out_specs=_split_spec,
    out_shape=jax.ShapeDtypeStruct((2, N, H), _f32),
)

_dense2 = pl.pallas_call(
    _dense2_kernel,
    grid=(N // R,),
    in_specs=_in_specs,
    out_specs=pl.BlockSpec((R, D), lambda i: (i, 0)),
    out_shape=jax.ShapeDtypeStruct((N, D), _f32),
)


def kernel(x, edge_index, W1_l, b1, W1_r, W2_l, b2, W2_r):
  src = edge_index[0]
  dst = edge_index[1]
  xs = x.reshape(N, 2, H).transpose(1, 0, 2)
  cnt = _count(dst)
  agg1 = _agg(xs, src, dst)
  h1 = _dense1(agg1, cnt, xs, W1_l, W1_r, b1.reshape(1, D))
  agg2 = _agg(h1, src, dst)
  return _dense2(agg2, cnt, h1, W2_l, W2_r, b2.reshape(1, D))


# speedup vs baseline: 6.7583x; 2.0527x over previous
import functools

import jax
import jax.numpy as jnp
from jax import lax
from jax.experimental import pallas as pl
from jax.experimental.pallas import tpu as pltpu
from jax.experimental.pallas import tpu_sc as plsc

N = 10000
E = 160000
D = 256
H = 128
NC, NS = 2, 16
KC = 128
EPAD = 163840
CPT = EPAD // (NS * KC)
CPT2 = EPAD // (NC * NS * KC)
NPAD = 16
RPT = 640
LAST = N - (NS - 1) * RPT
ZR = 40

_f32 = jnp.float32
_mesh = plsc.VectorSubcoreMesh(core_axis_name="c", subcore_axis_name="s")


@functools.partial(
    pl.kernel, mesh=_mesh,
    out_type=jax.ShapeDtypeStruct((NC, N, H), _f32),
    scratch_types=[
        pltpu.VMEM((1, KC), jnp.int32),
        pltpu.VMEM((1, KC), jnp.int32),
        pltpu.VMEM((1, KC), jnp.int32),
        pltpu.VMEM((1, KC), jnp.int32),
        pltpu.VMEM((KC, H), _f32),
        pltpu.VMEM((KC, H), _f32),
        pltpu.VMEM((ZR, H), _f32),
        pltpu.VMEM_SHARED((N + NPAD, H), _f32),
        pltpu.SemaphoreType.DMA,
        pltpu.SemaphoreType.DMA,
        pltpu.SemaphoreType.DMA,
        pltpu.SemaphoreType.DMA,
    ])
def _agg(table_hbm, src_hbm, dst_hbm, out_hbm,
         srcb0, srcb1, dstb0, dstb1, rows0, rows1, zrows, acc,
         g0, g1, i0sem, i1sem):
  cid = lax.axis_index("c")
  sid = lax.axis_index("s")
  z16 = jnp.zeros((1, 16), _f32)

  @pl.loop(0, ZR)
  def _(r):
    @pl.loop(0, H // 16)
    def _(c):
      zrows.at[pl.ds(r, 1), pl.ds(c * 16, 16)][...] = z16

  row0 = sid * RPT

  @pl.when(sid < NS - 1)
  def _():
    for j in range(RPT // ZR):
      pltpu.sync_copy(zrows, acc.at[pl.ds(row0 + j * ZR, ZR)])

  @pl.when(sid == NS - 1)
  def _():
    for j in range(LAST // ZR):
      pltpu.sync_copy(zrows, acc.at[pl.ds(row0 + j * ZR, ZR)])
    pltpu.sync_copy(zrows.at[pl.ds(0, NPAD)], acc.at[pl.ds(N, NPAD)])

  plsc.subcore_barrier()

  table = table_hbm.at[cid]
  c0 = sid * CPT

  def idx_fetch(i, sb, db, sem):
    pltpu.async_copy(src_hbm.at[pl.ds(c0 + i, 1)], sb, sem)
    pltpu.async_copy(dst_hbm.at[pl.ds(c0 + i, 1)], db, sem)

  def idx_wait(sb, db, sem):
    pltpu.make_async_copy(src_hbm.at[pl.ds(c0, 1)], sb, sem).wait()
    pltpu.make_async_copy(dst_hbm.at[pl.ds(c0, 1)], db, sem).wait()

  idx_fetch(0, srcb0, dstb0, i0sem)
  idx_wait(srcb0, dstb0, i0sem)
  pltpu.async_copy(table.at[srcb0.at[0]], rows0, g0)
  idx_fetch(1, srcb1, dstb1, i1sem)

  @pl.loop(0, CPT // 2)
  def _(j):
    i0 = 2 * j
    pltpu.make_async_copy(table.at[srcb0.at[0]], rows0, g0).wait()
    idx_wait(srcb1, dstb1, i1sem)
    pltpu.async_copy(table.at[srcb1.at[0]], rows1, g1)
    pltpu.sync_copy(rows0, acc.at[dstb0.at[0]], add=True)

    @pl.when(j < CPT // 2 - 1)
    def _():
      idx_fetch(i0 + 2, srcb0, dstb0, i0sem)

    pltpu.make_async_copy(table.at[srcb1.at[0]], rows1, g1).wait()

    @pl.when(j < CPT // 2 - 1)
    def _():
      idx_wait(srcb0, dstb0, i0sem)
      pltpu.async_copy(table.at[srcb0.at[0]], rows0, g0)

    pltpu.sync_copy(rows1, acc.at[dstb1.at[0]], add=True)

    @pl.when(j < CPT // 2 - 1)
    def _():
      idx_fetch(i0 + 3, srcb1, dstb1, i1sem)

  plsc.subcore_barrier()

  @pl.when(sid < NS - 1)
  def _():
    pltpu.sync_copy(acc.at[pl.ds(row0, RPT)],
                    out_hbm.at[cid].at[pl.ds(row0, RPT)])

  @pl.when(sid == NS - 1)
  def _():
    pltpu.sync_copy(acc.at[pl.ds(row0, LAST)],
                    out_hbm.at[cid].at[pl.ds(row0, LAST)])


@functools.partial(
    pl.kernel, mesh=_mesh,
    out_type=jax.ShapeDtypeStruct((NC, N, H), _f32),
    scratch_types=[
        pltpu.VMEM((CPT2, KC), jnp.int32),
        pltpu.VMEM((KC, H), _f32),
        pltpu.VMEM((ZR, H), _f32),
        pltpu.VMEM_SHARED((N + NPAD, H), _f32),
        pltpu.SemaphoreType.DMA,
    ])
def _count(dst_hbm, out_hbm, dstb, oneb, zrows, cntacc, ssem):
  cid = lax.axis_index("c")
  sid = lax.axis_index("s")
  z16 = jnp.zeros((1, 16), _f32)
  one16 = jnp.ones((1, 16), _f32)

  @pl.loop(0, ZR)
  def _(r):
    @pl.loop(0, H // 16)
    def _(c):
      zrows.at[pl.ds(r, 1), pl.ds(c * 16, 16)][...] = z16

  @pl.loop(0, KC)
  def _(r):
    @pl.loop(0, H // 16)
    def _(c):
      oneb.at[pl.ds(r, 1), pl.ds(c * 16, 16)][...] = one16

  row0 = sid * RPT

  @pl.when(sid < NS - 1)
  def _():
    for j in range(RPT // ZR):
      pltpu.sync_copy(zrows, cntacc.at[pl.ds(row0 + j * ZR, ZR)])

  @pl.when(sid == NS - 1)
  def _():
    for j in range(LAST // ZR):
      pltpu.sync_copy(zrows, cntacc.at[pl.ds(row0 + j * ZR, ZR)])
    pltpu.sync_copy(zrows.at[pl.ds(0, NPAD)], cntacc.at[pl.ds(N, NPAD)])

  pltpu.sync_copy(dst_hbm.at[pl.ds((cid * NS + sid) * CPT2, CPT2)], dstb)

  plsc.subcore_barrier()

  @pl.loop(0, CPT2)
  def _(i):
    pltpu.async_copy(oneb, cntacc.at[dstb.at[i]], add=True, sem=ssem)

  @pl.loop(0, CPT2)
  def _(i):
    pltpu.make_async_copy(oneb, cntacc.at[dstb.at[0]], ssem).wait()

  plsc.subcore_barrier()

  @pl.when(sid < NS - 1)
  def _():
    pltpu.sync_copy(cntacc.at[pl.ds(row0, RPT)],
                    out_hbm.at[cid].at[pl.ds(row0, RPT)])

  @pl.when(sid == NS - 1)
  def _():
    pltpu.sync_copy(cntacc.at[pl.ds(row0, LAST)],
                    out_hbm.at[cid].at[pl.ds(row0, LAST)])


R = 400
_CT = (((1,), (1,)), ((), ()))


def _dense_body(agg_ref, cnt_ref, x_ref, wl_ref, wr_ref, b_ref):
  cnt = cnt_ref[0][:, 0:1] + cnt_ref[1][:, 0:1]
  inv = 1.0 / jnp.maximum(cnt, 1.0)
  wl = wl_ref[...]
  wr = wr_ref[...]
  z = lax.dot_general(agg_ref[0] * inv, wl[:, :H], _CT,
                      preferred_element_type=_f32)
  z += lax.dot_general(agg_ref[1] * inv, wl[:, H:], _CT,
                       preferred_element_type=_f32)
  z += lax.dot_general(x_ref[0], wr[:, :H], _CT, preferred_element_type=_f32)
  z += lax.dot_general(x_ref[1], wr[:, H:], _CT, preferred_element_type=_f32)
  return z + b_ref[...]


def _dense1_kernel(agg_ref, cnt_ref, x_ref, wl_ref, wr_ref, b_ref, out_ref):
  h = jnp.maximum(_dense_body(agg_ref, cnt_ref, x_ref, wl_ref, wr_ref, b_ref),
                  0.0)
  out_ref[0] = h[:, :H]
  out_ref[1] = h[:, H:]


def _dense2_kernel(agg_ref, cnt_ref, x_ref, wl_ref, wr_ref, b_ref, out_ref):
  z = _dense_body(agg_ref, cnt_ref, x_ref, wl_ref, wr_ref, b_ref)
  nrm = jnp.sqrt(jnp.sum(z * z, axis=1, keepdims=True))
  out_ref[...] = z / jnp.maximum(nrm, 1e-12)


_split_spec = pl.BlockSpec((2, R, H), lambda i: (0, i, 0))
_in_specs = [
    _split_spec,
    _split_spec,
    _split_spec,
    pl.BlockSpec((D, D), lambda i: (0, 0)),
    pl.BlockSpec((D, D), lambda i: (0, 0)),
    pl.BlockSpec((1, D), lambda i: (0, 0)),
]

_dense1 = pl.pallas_call(
    _dense1_kernel,
    grid=(N // R,),
    in_specs=_in_specs,
    out_specs=_split_spec,
    out_shape=jax.ShapeDtypeStruct((2, N, H), _f32),
)

_dense2 = pl.pallas_call(
    _dense2_kernel,
    grid=(N // R,),
    in_specs=_in_specs,
    out_specs=pl.BlockSpec((R, D), lambda i: (i, 0)),
    out_shape=jax.ShapeDtypeStruct((N, D), _f32),
)


def kernel(x, edge_index, W1_l, b1, W1_r, W2_l, b2, W2_r):
  src = edge_index[0]
  dst = edge_index[1]
  pad = jnp.arange(EPAD - E, dtype=jnp.int32)
  srcp = jnp.concatenate([src, pad % N]).reshape(NS * CPT, KC)
  dstp = jnp.concatenate([dst, N + (pad % NPAD)]).reshape(NS * CPT, KC)
  xs = x.reshape(N, 2, H).transpose(1, 0, 2)
  cnt = _count(dstp)
  agg1 = _agg(xs, srcp, dstp)
  h1 = _dense1(agg1, cnt, xs, W1_l, W1_r, b1.reshape(1, D))
  agg2 = _agg(h1, srcp, dstp)
  return _dense2(agg2, cnt, h1, W2_l, W2_r, b2.reshape(1, D))
